# baseline (speedup 1.0000x reference)
import functools

import jax
import jax.numpy as jnp
from jax import lax
from jax.experimental import pallas as pl
from jax.experimental.pallas import tpu as pltpu
from jax.experimental.pallas import tpu_sc as plsc

_B = 32
_CHUNK = 128
_CANVAS = 8192
_D = 256
_ALIGN = 8
_WIN = _CHUNK + _ALIGN


def _sc_build_merged(value, index, cache):
    mesh = plsc.VectorSubcoreMesh(core_axis_name="c", subcore_axis_name="s")

    @functools.partial(
        pl.kernel,
        mesh=mesh,
        out_type=jax.ShapeDtypeStruct((_B, _WIN, _D), jnp.float32),
        scratch_types=[
            pltpu.VMEM((_WIN, _D), jnp.float32),
            pltpu.VMEM((_CHUNK, _D), jnp.float32),
            pltpu.VMEM((16,), jnp.int32),
            pltpu.SemaphoreType.DMA,
        ],
    )
    def merge(value_hbm, index_hbm, cache_hbm, merged_hbm,
              win, val, idx_v, sem):
        wid = lax.axis_index("s") * 2 + lax.axis_index("c")
        b = wid

        pltpu.sync_copy(index_hbm, idx_v.at[pl.ds(0, 1)])
        idx = idx_v[...][0]
        base = pl.multiple_of((idx // _ALIGN) * _ALIGN, _ALIGN)
        off = idx - base

        head = cache_hbm.at[b, pl.ds(base, _ALIGN), :]
        tail = cache_hbm.at[b, pl.ds(base + _CHUNK, _ALIGN), :]
        pltpu.async_copy(head, win.at[pl.ds(0, _ALIGN)], sem)
        pltpu.async_copy(tail, win.at[pl.ds(_CHUNK, _ALIGN)], sem)
        pltpu.async_copy(value_hbm.at[b], val, sem)
        pltpu.make_async_copy(head, win.at[pl.ds(0, _ALIGN)], sem).wait()
        pltpu.make_async_copy(tail, win.at[pl.ds(_CHUNK, _ALIGN)], sem).wait()
        pltpu.make_async_copy(value_hbm.at[b], val, sem).wait()

        @plsc.parallel_loop(0, _CHUNK, unroll=4)
        def _splice(r):
            for c in range(_D // 16):
                win[off + r, pl.ds(c * 16, 16)] = val[r, pl.ds(c * 16, 16)]

        pltpu.sync_copy(win, merged_hbm.at[b])

    return merge(value, index, cache)


def _copy_splice_kernel(index_ref, merged_ref, in_ref, out_ref):
    idx = index_ref[0]
    base = pl.multiple_of((idx // _ALIGN) * _ALIGN, _ALIGN)
    out_ref[...] = in_ref[...]
    out_ref[0, pl.ds(base, _WIN), :] = merged_ref[0]


def _tc_copy_splice(index, merged, cache):
    grid_spec = pltpu.PrefetchScalarGridSpec(
        num_scalar_prefetch=1,
        grid=(_B,),
        in_specs=[
            pl.BlockSpec((1, _WIN, _D), lambda b, idx: (b, 0, 0)),
            pl.BlockSpec((1, _CANVAS, _D), lambda b, idx: (b, 0, 0)),
        ],
        out_specs=pl.BlockSpec((1, _CANVAS, _D), lambda b, idx: (b, 0, 0)),
    )
    return pl.pallas_call(
        _copy_splice_kernel,
        grid_spec=grid_spec,
        out_shape=jax.ShapeDtypeStruct((_B, _CANVAS, _D), cache.dtype),
    )(index, merged, cache)


def kernel(value, index, cache):
    merged = _sc_build_merged(value, index, cache)
    return _tc_copy_splice(index, merged, cache)


# speedup vs baseline: 1.0019x; 1.0019x over previous
import functools

import jax
import jax.numpy as jnp
from jax import lax
from jax.experimental import pallas as pl
from jax.experimental.pallas import tpu as pltpu
from jax.experimental.pallas import tpu_sc as plsc

_B = 32
_CHUNK = 128
_CANVAS = 8192
_D = 256
_ALIGN = 8
_WIN = _CHUNK + _ALIGN


def _sc_shift_value(value, index):
    mesh = plsc.VectorSubcoreMesh(core_axis_name="c", subcore_axis_name="s")

    @functools.partial(
        pl.kernel,
        mesh=mesh,
        out_type=jax.ShapeDtypeStruct((_B, _WIN, _D), jnp.float32),
        scratch_types=[
            pltpu.VMEM((_WIN, _D), jnp.float32),
            pltpu.VMEM((_CHUNK, _D), jnp.float32),
            pltpu.VMEM((16,), jnp.int32),
        ],
    )
    def shift(value_hbm, index_hbm, shifted_hbm, win, val, idx_v):
        wid = lax.axis_index("s") * 2 + lax.axis_index("c")
        b = wid

        pltpu.sync_copy(index_hbm, idx_v.at[pl.ds(0, 1)])
        idx = idx_v[...][0]
        off = lax.rem(idx, _ALIGN)

        pltpu.sync_copy(value_hbm.at[b], val)

        @plsc.parallel_loop(0, _CHUNK, unroll=4)
        def _splice(r):
            for c in range(_D // 16):
                win[off + r, pl.ds(c * 16, 16)] = val[r, pl.ds(c * 16, 16)]

        pltpu.sync_copy(win, shifted_hbm.at[b])

    return shift(value, index)


def _copy_splice_kernel(index_ref, shifted_ref, in_ref, out_ref):
    idx = index_ref[0]
    base = pl.multiple_of((idx // _ALIGN) * _ALIGN, _ALIGN)
    off = idx - base
    out_ref[...] = in_ref[...]
    r = jax.lax.broadcasted_iota(jnp.int32, (1, _WIN, _D), 1)
    mask = jnp.logical_and(r >= off, r < off + _CHUNK)
    sub = in_ref[0, pl.ds(base, _WIN), :][jnp.newaxis]
    out_ref[0, pl.ds(base, _WIN), :] = \
        jnp.where(mask, shifted_ref[...], sub)[0]


def _tc_copy_splice(index, shifted, cache):
    grid_spec = pltpu.PrefetchScalarGridSpec(
        num_scalar_prefetch=1,
        grid=(_B,),
        in_specs=[
            pl.BlockSpec((1, _WIN, _D), lambda b, idx: (b, 0, 0)),
            pl.BlockSpec((1, _CANVAS, _D), lambda b, idx: (b, 0, 0)),
        ],
        out_specs=pl.BlockSpec((1, _CANVAS, _D), lambda b, idx: (b, 0, 0)),
    )
    return pl.pallas_call(
        _copy_splice_kernel,
        grid_spec=grid_spec,
        out_shape=jax.ShapeDtypeStruct((_B, _CANVAS, _D), cache.dtype),
    )(index, shifted, cache)


def kernel(value, index, cache):
    shifted = _sc_shift_value(value, index)
    return _tc_copy_splice(index, shifted, cache)


# speedup vs baseline: 1.0084x; 1.0065x over previous
import functools

import jax
import jax.numpy as jnp
from jax import lax
from jax.experimental import pallas as pl
from jax.experimental.pallas import tpu as pltpu
from jax.experimental.pallas import tpu_sc as plsc

_B = 32
_CHUNK = 128
_CANVAS = 8192
_D = 256
_ALIGN = 8
_WIN = _CHUNK + _ALIGN


def _sc_build_merged(value, index, cache):
    mesh = plsc.VectorSubcoreMesh(core_axis_name="c", subcore_axis_name="s")

    @functools.partial(
        pl.kernel,
        mesh=mesh,
        out_type=jax.ShapeDtypeStruct((_B, _WIN, _D), jnp.float32),
        cost_estimate=pl.CostEstimate(
            flops=4_000_000, bytes_accessed=14_000_000, transcendentals=0),
        scratch_types=[
            pltpu.VMEM((_WIN, _D), jnp.float32),
            pltpu.VMEM((_CHUNK, _D), jnp.float32),
            pltpu.VMEM((16,), jnp.int32),
            pltpu.SemaphoreType.DMA,
        ],
    )
    def merge(value_hbm, index_hbm, cache_hbm, merged_hbm,
              win, val, idx_v, sem):
        wid = lax.axis_index("s") * 2 + lax.axis_index("c")
        b = wid

        pltpu.sync_copy(index_hbm, idx_v.at[pl.ds(0, 1)])
        idx = idx_v[...][0]
        base = pl.multiple_of((idx // _ALIGN) * _ALIGN, _ALIGN)
        off = idx - base

        pltpu.async_copy(cache_hbm.at[b, pl.ds(base, _WIN), :], win, sem)
        pltpu.async_copy(value_hbm.at[b], val, sem)
        pltpu.make_async_copy(
            cache_hbm.at[b, pl.ds(base, _WIN), :], win, sem).wait()
        pltpu.make_async_copy(value_hbm.at[b], val, sem).wait()

        def splice(r, carry):
            for c in range(_D // 16):
                win[off + r, pl.ds(c * 16, 16)] = val[r, pl.ds(c * 16, 16)]
            return carry

        lax.fori_loop(0, _CHUNK, splice, 0)
        pltpu.sync_copy(win, merged_hbm.at[b])

    return merge(value, index, cache)


def _copy_kernel(in_ref, out_ref):
    out_ref[...] = in_ref[...]


def _tc_bulk_copy(cache):
    return pl.pallas_call(
        _copy_kernel,
        grid=(_B,),
        in_specs=[pl.BlockSpec((1, _CANVAS, _D), lambda b: (b, 0, 0))],
        out_specs=pl.BlockSpec((1, _CANVAS, _D), lambda b: (b, 0, 0)),
        out_shape=jax.ShapeDtypeStruct((_B, _CANVAS, _D), cache.dtype),
        cost_estimate=pl.CostEstimate(
            flops=0, bytes_accessed=536_870_912, transcendentals=0),
    )(cache)


def _splice_kernel(index_ref, merged_ref, outin_ref, out_ref, sem):
    del outin_ref
    idx = index_ref[0]
    base = pl.multiple_of((idx // _ALIGN) * _ALIGN, _ALIGN)
    cp = pltpu.make_async_copy(
        merged_ref, out_ref.at[:, pl.ds(base, _WIN), :], sem)
    cp.start()
    cp.wait()


def _tc_splice(index, merged, out):
    return pl.pallas_call(
        _splice_kernel,
        in_specs=[
            pl.BlockSpec(memory_space=pltpu.SMEM),
            pl.BlockSpec(memory_space=pltpu.VMEM),
            pl.BlockSpec(memory_space=pl.ANY),
        ],
        out_specs=pl.BlockSpec(memory_space=pl.ANY),
        out_shape=jax.ShapeDtypeStruct((_B, _CANVAS, _D), out.dtype),
        input_output_aliases={2: 0},
        scratch_shapes=[pltpu.SemaphoreType.DMA],
    )(index, merged, out)


def kernel(value, index, cache):
    merged = _sc_build_merged(value, index, cache)
    out = _tc_bulk_copy(cache)
    return _tc_splice(index, merged, out)
